# Initial kernel scaffold; baseline (speedup 1.0000x reference)
import jax
import jax.numpy as jnp
from jax.experimental import pallas as pl


def kernel(x, edge_index, params):
    raise NotImplementedError("write your pallas kernel here")


# speedup vs baseline: 8.2940x; 8.2940x over previous
import functools

import jax
import jax.numpy as jnp
from jax import lax
from jax.experimental import pallas as pl
from jax.experimental.pallas import tpu as pltpu
from jax.experimental.pallas import tpu_sc as plsc

BN_EPS = 1e-5
NC = 2
NS = 16
NW = NC * NS
CHUNK = 80
HIGH = lax.Precision.HIGHEST


def _sc_body(ngrp, nchg, h_hbm, src_hbm, dst_hbm, out_hbm,
             sidx, didx, rows, acc, gsem):
    n = out_hbm.shape[1]
    d = h_hbm.shape[1]
    cid = lax.axis_index("c")
    sid = lax.axis_index("s")
    wid = sid * NC + cid
    zr = rows.shape[1]
    ncopies = n // zr

    def zrow(r, carry):
        def zcol(c, carry2):
            rows[0, r, pl.ds(c * 16, 16)] = jnp.zeros((16,), jnp.float32)
            return carry2
        return lax.fori_loop(0, d // 16, zcol, carry)
    lax.fori_loop(0, zr, zrow, 0)

    for k in range(-(-ncopies // NS)):
        j = sid + k * NS

        @pl.when(j < ncopies)
        def _():
            pltpu.sync_copy(rows.at[0],
                            acc.at[pl.ds(pl.multiple_of(j * zr, 8), zr)])
    plsc.subcore_barrier()

    def gather(i, b):
        pltpu.async_copy(h_hbm.at[sidx.at[i]], rows.at[b], gsem.at[b])

    def drain(i, b):
        pltpu.make_async_copy(h_hbm.at[sidx.at[i]], rows.at[b],
                              gsem.at[b]).wait()
        pltpu.sync_copy(rows.at[b], acc.at[didx.at[i]], add=True)

    for g in range(ngrp):
        pltpu.sync_copy(src_hbm.at[wid, g], sidx)
        pltpu.sync_copy(dst_hbm.at[wid, g], didx)
        gather(0, 0)

        def step(j, carry):
            i0 = j * 2

            @pl.when(i0 + 1 < nchg)
            def _():
                gather(i0 + 1, 1)
            drain(i0, 0)

            @pl.when(i0 + 2 < nchg)
            def _():
                gather(i0 + 2, 0)

            @pl.when(i0 + 1 < nchg)
            def _():
                drain(i0 + 1, 1)
            return carry

        lax.fori_loop(0, (nchg + 1) // 2, step, 0)

    plsc.subcore_barrier()

    for k in range(-(-ncopies // NS)):
        j = sid + k * NS

        @pl.when(j < ncopies)
        def _():
            sl = pl.ds(pl.multiple_of(j * zr, 8), zr)
            pltpu.sync_copy(acc.at[sl], out_hbm.at[cid, sl])


def _sc_segment_sum(h, src4, dst4):
    n, d = h.shape
    _, ngrp, nchg, c = src4.shape
    mesh = plsc.VectorSubcoreMesh(core_axis_name="c", subcore_axis_name="s")
    f = pl.kernel(
        functools.partial(_sc_body, ngrp, nchg),
        out_type=jax.ShapeDtypeStruct((NC, n, d), jnp.float32),
        mesh=mesh,
        scratch_types=[
            pltpu.VMEM((nchg, c), jnp.int32),
            pltpu.VMEM((nchg, c), jnp.int32),
            pltpu.VMEM((2, c, d), jnp.float32),
            pltpu.VMEM_SHARED((n, d), jnp.float32),
            pltpu.SemaphoreType.DMA((2,)),
        ],
    )
    return f(h, src4, dst4)


def _body_a(h_ref, p0_ref, p1_ref, w1_ref, t_ref, s_ref):
    r = h_ref[...] + p0_ref[...] + p1_ref[...]
    t = lax.dot_general(r, w1_ref[...], (((1,), (1,)), ((), ())),
                        precision=HIGH)
    t_ref[...] = t
    s0 = jnp.sum(t, axis=0, keepdims=True)
    s1 = jnp.sum(t * t, axis=0, keepdims=True)
    blk = jnp.concatenate(
        [s0, s1, jnp.zeros((6, t.shape[1]), jnp.float32)], axis=0)

    @pl.when(pl.program_id(0) == 0)
    def _():
        s_ref[...] = jnp.zeros_like(s_ref)
    s_ref[...] += blk


def _bn_scale_shift(s_ref, g_ref, b_ref, n):
    m = s_ref[pl.ds(0, 1), :] * (1.0 / n)
    ex2 = s_ref[pl.ds(1, 1), :] * (1.0 / n)
    v = ex2 - m * m
    scale = g_ref[...] * lax.rsqrt(v + BN_EPS)
    shift = b_ref[...] - m * scale
    return scale, shift


def _body_b(n, t_ref, s_ref, g_ref, b_ref, w2_ref, o_ref, s2_ref):
    scale, shift = _bn_scale_shift(s_ref, g_ref, b_ref, n)
    u = jnp.maximum(t_ref[...] * scale + shift, 0.0)
    o = lax.dot_general(u, w2_ref[...], (((1,), (1,)), ((), ())),
                        precision=HIGH)
    o_ref[...] = o
    s0 = jnp.sum(o, axis=0, keepdims=True)
    s1 = jnp.sum(o * o, axis=0, keepdims=True)
    blk = jnp.concatenate(
        [s0, s1, jnp.zeros((6, o.shape[1]), jnp.float32)], axis=0)

    @pl.when(pl.program_id(0) == 0)
    def _():
        s2_ref[...] = jnp.zeros_like(s2_ref)
    s2_ref[...] += blk


def _body_c(n, o_ref, s2_ref, g_ref, b_ref, h_ref, p_ref):
    scale, shift = _bn_scale_shift(s2_ref, g_ref, b_ref, n)
    hh = jnp.maximum(o_ref[...] * scale + shift, 0.0)
    h_ref[...] = hh
    blk = jnp.concatenate(
        [jnp.sum(hh, axis=0, keepdims=True),
         jnp.zeros((7, hh.shape[1]), jnp.float32)], axis=0)

    @pl.when(pl.program_id(0) == 0)
    def _():
        p_ref[...] = jnp.zeros_like(p_ref)
    p_ref[...] += blk


def _body_pool(x_ref, p_ref):
    blk = jnp.concatenate(
        [jnp.sum(x_ref[...], axis=0, keepdims=True),
         jnp.zeros((7, x_ref.shape[1]), jnp.float32)], axis=0)

    @pl.when(pl.program_id(0) == 0)
    def _():
        p_ref[...] = jnp.zeros_like(p_ref)
    p_ref[...] += blk


def _body_head(pall_ref, pw_ref, pb_ref, out_ref):
    d = out_ref.shape[-1]
    acc = jnp.zeros((1, d), jnp.float32)
    for i in range(5):
        p = pall_ref[pl.ds(i, 1), :]
        w = pw_ref[pl.ds(i * d, d), :]
        acc = acc + lax.dot_general(p, w, (((1,), (1,)), ((), ())),
                                    precision=HIGH) + pb_ref[pl.ds(i, 1), :]
    z = acc - jnp.max(acc, axis=-1, keepdims=True)
    out_ref[...] = z - jnp.log(jnp.sum(jnp.exp(z), axis=-1, keepdims=True))


def _row_spec(r, d):
    return pl.BlockSpec((r, d), lambda i: (i, 0))


def _fix_spec(shape):
    return pl.BlockSpec(shape, lambda i: tuple(0 for _ in shape))


def _tc_layer(h, p0, p1, w1, g1, b1, w2, g2, b2):
    n, d = h.shape
    r = 1000
    g = n // r
    f32 = jnp.float32
    t, s1 = pl.pallas_call(
        _body_a, grid=(g,),
        in_specs=[_row_spec(r, d)] * 3 + [_fix_spec((d, d))],
        out_specs=[_row_spec(r, d), _fix_spec((8, d))],
        out_shape=[jax.ShapeDtypeStruct((n, d), f32),
                   jax.ShapeDtypeStruct((8, d), f32)],
    )(h, p0, p1, w1)
    o, s2 = pl.pallas_call(
        functools.partial(_body_b, n), grid=(g,),
        in_specs=[_row_spec(r, d), _fix_spec((8, d)), _fix_spec((1, d)),
                  _fix_spec((1, d)), _fix_spec((d, d))],
        out_specs=[_row_spec(r, d), _fix_spec((8, d))],
        out_shape=[jax.ShapeDtypeStruct((n, d), f32),
                   jax.ShapeDtypeStruct((8, d), f32)],
    )(t, s1, g1, b1, w2)
    hh, pool = pl.pallas_call(
        functools.partial(_body_c, n), grid=(g,),
        in_specs=[_row_spec(r, d), _fix_spec((8, d)), _fix_spec((1, d)),
                  _fix_spec((1, d))],
        out_specs=[_row_spec(r, d), _fix_spec((8, d))],
        out_shape=[jax.ShapeDtypeStruct((n, d), f32),
                   jax.ShapeDtypeStruct((8, d), f32)],
    )(o, s2, g2, b2)
    return hh, pool


def _pool_sum(x):
    n, d = x.shape
    r = 1000
    return pl.pallas_call(
        _body_pool, grid=(n // r,),
        in_specs=[_row_spec(r, d)],
        out_specs=_fix_spec((8, d)),
        out_shape=jax.ShapeDtypeStruct((8, d), jnp.float32),
    )(x)


def _head(pall, pw, pb):
    d = pall.shape[1]
    return pl.pallas_call(
        _body_head,
        in_specs=[pl.BlockSpec((8, d), None),
                  pl.BlockSpec((5 * d, d), None),
                  pl.BlockSpec((8, d), None)],
        out_specs=pl.BlockSpec((1, d), None),
        out_shape=jax.ShapeDtypeStruct((1, d), jnp.float32),
    )(pall, pw, pb)


def kernel(x, edge_index, params):
    n, d = x.shape
    e = edge_index.shape[1]
    ngrp = 5
    nchg = e // (NW * CHUNK * ngrp)
    src4 = edge_index[0].reshape(NW, ngrp, nchg, CHUNK)
    dst4 = edge_index[1].reshape(NW, ngrp, nchg, CHUNK)

    h = x
    pools = []
    for i in range(4):
        agg = _sc_segment_sum(h, src4, dst4)
        h, pool = _tc_layer(
            h, agg[0], agg[1],
            params["gin_W1"][i],
            params["gin_bn_g"][i].reshape(1, d),
            params["gin_bn_b"][i].reshape(1, d),
            params["gin_W2"][i],
            params["bn_g"][i].reshape(1, d),
            params["bn_b"][i].reshape(1, d),
        )
        pools.append(pool)

    px = _pool_sum(x)
    zeros3 = jnp.zeros((3, d), jnp.float32)
    pall = jnp.concatenate([px[0:1]] + [p[0:1] for p in pools] + [zeros3], 0)
    pw = jnp.concatenate(list(params["pred_W"]), axis=0)
    pb = jnp.concatenate([b.reshape(1, d) for b in params["pred_b"]]
                         + [zeros3], axis=0)
    return _head(pall, pw, pb)


# speedup vs baseline: 8.7380x; 1.0535x over previous
import functools

import jax
import jax.numpy as jnp
from jax import lax
from jax.experimental import pallas as pl
from jax.experimental.pallas import tpu as pltpu
from jax.experimental.pallas import tpu_sc as plsc

BN_EPS = 1e-5
NC = 2
NS = 16
NW = NC * NS
CHUNK = 125
HIGH = lax.Precision.HIGHEST


def _sc_body(ngrp, nchg, h_hbm, src_hbm, dst_hbm, out_hbm,
             sidx, didx, rows, acc, gsem):
    n = out_hbm.shape[1]
    d = h_hbm.shape[1]
    cid = lax.axis_index("c")
    sid = lax.axis_index("s")
    wid = sid * NC + cid
    zr = 80
    ncopies = n // zr

    def zrow(r, carry):
        def zcol(c, carry2):
            rows[0, r, pl.ds(c * 16, 16)] = jnp.zeros((16,), jnp.float32)
            return carry2
        return lax.fori_loop(0, d // 16, zcol, carry)
    lax.fori_loop(0, zr, zrow, 0)

    for k in range(-(-ncopies // NS)):
        j = sid + k * NS

        @pl.when(j < ncopies)
        def _():
            pltpu.sync_copy(rows.at[0, pl.ds(0, zr)],
                            acc.at[pl.ds(pl.multiple_of(j * zr, 8), zr)])
    plsc.subcore_barrier()

    def gather(i, b):
        pltpu.async_copy(h_hbm.at[sidx.at[i]], rows.at[b], gsem.at[b])

    def drain(i, b):
        pltpu.make_async_copy(h_hbm.at[sidx.at[i]], rows.at[b],
                              gsem.at[b]).wait()
        pltpu.sync_copy(rows.at[b], acc.at[didx.at[i]], add=True)

    for g in range(ngrp):
        pltpu.sync_copy(src_hbm.at[wid, g], sidx)
        pltpu.sync_copy(dst_hbm.at[wid, g], didx)
        gather(0, 0)

        def step(j, carry):
            i0 = j * 2

            @pl.when(i0 + 1 < nchg)
            def _():
                gather(i0 + 1, 1)
            drain(i0, 0)

            @pl.when(i0 + 2 < nchg)
            def _():
                gather(i0 + 2, 0)

            @pl.when(i0 + 1 < nchg)
            def _():
                drain(i0 + 1, 1)
            return carry

        lax.fori_loop(0, (nchg + 1) // 2, step, 0)

    plsc.subcore_barrier()

    for k in range(-(-ncopies // NS)):
        j = sid + k * NS

        @pl.when(j < ncopies)
        def _():
            sl = pl.ds(pl.multiple_of(j * zr, 8), zr)
            pltpu.sync_copy(acc.at[sl], out_hbm.at[cid, sl])


def _sc_segment_sum(h, src4, dst4):
    n, d = h.shape
    _, ngrp, nchg, c = src4.shape
    mesh = plsc.VectorSubcoreMesh(core_axis_name="c", subcore_axis_name="s")
    f = pl.kernel(
        functools.partial(_sc_body, ngrp, nchg),
        out_type=jax.ShapeDtypeStruct((NC, n, d), jnp.float32),
        mesh=mesh,
        scratch_types=[
            pltpu.VMEM((nchg, c), jnp.int32),
            pltpu.VMEM((nchg, c), jnp.int32),
            pltpu.VMEM((2, c, d), jnp.float32),
            pltpu.VMEM_SHARED((n, d), jnp.float32),
            pltpu.SemaphoreType.DMA((2,)),
        ],
    )
    return f(h, src4, dst4)


def _body_a(h_ref, p0_ref, p1_ref, w1_ref, t_ref, s_ref):
    r = h_ref[...] + p0_ref[...] + p1_ref[...]
    t = lax.dot_general(r, w1_ref[...], (((1,), (1,)), ((), ())),
                        precision=HIGH)
    t_ref[...] = t
    s0 = jnp.sum(t, axis=0, keepdims=True)
    s1 = jnp.sum(t * t, axis=0, keepdims=True)
    blk = jnp.concatenate(
        [s0, s1, jnp.zeros((6, t.shape[1]), jnp.float32)], axis=0)

    @pl.when(pl.program_id(0) == 0)
    def _():
        s_ref[...] = jnp.zeros_like(s_ref)
    s_ref[...] += blk


def _bn_scale_shift(s_ref, g_ref, b_ref, n):
    m = s_ref[pl.ds(0, 1), :] * (1.0 / n)
    ex2 = s_ref[pl.ds(1, 1), :] * (1.0 / n)
    v = ex2 - m * m
    scale = g_ref[...] * lax.rsqrt(v + BN_EPS)
    shift = b_ref[...] - m * scale
    return scale, shift


def _body_b(n, t_ref, s_ref, g_ref, b_ref, w2_ref, o_ref, s2_ref):
    scale, shift = _bn_scale_shift(s_ref, g_ref, b_ref, n)
    u = jnp.maximum(t_ref[...] * scale + shift, 0.0)
    o = lax.dot_general(u, w2_ref[...], (((1,), (1,)), ((), ())),
                        precision=HIGH)
    o_ref[...] = o
    s0 = jnp.sum(o, axis=0, keepdims=True)
    s1 = jnp.sum(o * o, axis=0, keepdims=True)
    blk = jnp.concatenate(
        [s0, s1, jnp.zeros((6, o.shape[1]), jnp.float32)], axis=0)

    @pl.when(pl.program_id(0) == 0)
    def _():
        s2_ref[...] = jnp.zeros_like(s2_ref)
    s2_ref[...] += blk


def _body_c(n, o_ref, s2_ref, g_ref, b_ref, h_ref, p_ref):
    scale, shift = _bn_scale_shift(s2_ref, g_ref, b_ref, n)
    hh = jnp.maximum(o_ref[...] * scale + shift, 0.0)
    h_ref[...] = hh
    blk = jnp.concatenate(
        [jnp.sum(hh, axis=0, keepdims=True),
         jnp.zeros((7, hh.shape[1]), jnp.float32)], axis=0)

    @pl.when(pl.program_id(0) == 0)
    def _():
        p_ref[...] = jnp.zeros_like(p_ref)
    p_ref[...] += blk


def _body_pool(x_ref, p_ref):
    blk = jnp.concatenate(
        [jnp.sum(x_ref[...], axis=0, keepdims=True),
         jnp.zeros((7, x_ref.shape[1]), jnp.float32)], axis=0)

    @pl.when(pl.program_id(0) == 0)
    def _():
        p_ref[...] = jnp.zeros_like(p_ref)
    p_ref[...] += blk


def _body_head(pall_ref, pw_ref, pb_ref, out_ref):
    d = out_ref.shape[-1]
    acc = jnp.zeros((1, d), jnp.float32)
    for i in range(5):
        p = pall_ref[pl.ds(i, 1), :]
        w = pw_ref[pl.ds(i * d, d), :]
        acc = acc + lax.dot_general(p, w, (((1,), (1,)), ((), ())),
                                    precision=HIGH) + pb_ref[pl.ds(i, 1), :]
    z = acc - jnp.max(acc, axis=-1, keepdims=True)
    out_ref[...] = z - jnp.log(jnp.sum(jnp.exp(z), axis=-1, keepdims=True))


def _row_spec(r, d):
    return pl.BlockSpec((r, d), lambda i: (i, 0))


def _fix_spec(shape):
    return pl.BlockSpec(shape, lambda i: tuple(0 for _ in shape))


def _tc_layer(h, p0, p1, w1, g1, b1, w2, g2, b2):
    n, d = h.shape
    r = 1000
    g = n // r
    f32 = jnp.float32
    t, s1 = pl.pallas_call(
        _body_a, grid=(g,),
        in_specs=[_row_spec(r, d)] * 3 + [_fix_spec((d, d))],
        out_specs=[_row_spec(r, d), _fix_spec((8, d))],
        out_shape=[jax.ShapeDtypeStruct((n, d), f32),
                   jax.ShapeDtypeStruct((8, d), f32)],
    )(h, p0, p1, w1)
    o, s2 = pl.pallas_call(
        functools.partial(_body_b, n), grid=(g,),
        in_specs=[_row_spec(r, d), _fix_spec((8, d)), _fix_spec((1, d)),
                  _fix_spec((1, d)), _fix_spec((d, d))],
        out_specs=[_row_spec(r, d), _fix_spec((8, d))],
        out_shape=[jax.ShapeDtypeStruct((n, d), f32),
                   jax.ShapeDtypeStruct((8, d), f32)],
    )(t, s1, g1, b1, w2)
    hh, pool = pl.pallas_call(
        functools.partial(_body_c, n), grid=(g,),
        in_specs=[_row_spec(r, d), _fix_spec((8, d)), _fix_spec((1, d)),
                  _fix_spec((1, d))],
        out_specs=[_row_spec(r, d), _fix_spec((8, d))],
        out_shape=[jax.ShapeDtypeStruct((n, d), f32),
                   jax.ShapeDtypeStruct((8, d), f32)],
    )(o, s2, g2, b2)
    return hh, pool


def _pool_sum(x):
    n, d = x.shape
    r = 1000
    return pl.pallas_call(
        _body_pool, grid=(n // r,),
        in_specs=[_row_spec(r, d)],
        out_specs=_fix_spec((8, d)),
        out_shape=jax.ShapeDtypeStruct((8, d), jnp.float32),
    )(x)


def _head(pall, pw, pb):
    d = pall.shape[1]
    return pl.pallas_call(
        _body_head,
        in_specs=[pl.BlockSpec((8, d), None),
                  pl.BlockSpec((5 * d, d), None),
                  pl.BlockSpec((8, d), None)],
        out_specs=pl.BlockSpec((1, d), None),
        out_shape=jax.ShapeDtypeStruct((1, d), jnp.float32),
    )(pall, pw, pb)


def kernel(x, edge_index, params):
    n, d = x.shape
    e = edge_index.shape[1]
    ngrp = 5
    nchg = e // (NW * CHUNK * ngrp)
    src4 = edge_index[0].reshape(NW, ngrp, nchg, CHUNK)
    dst4 = edge_index[1].reshape(NW, ngrp, nchg, CHUNK)

    h = x
    pools = []
    for i in range(4):
        agg = _sc_segment_sum(h, src4, dst4)
        h, pool = _tc_layer(
            h, agg[0], agg[1],
            params["gin_W1"][i],
            params["gin_bn_g"][i].reshape(1, d),
            params["gin_bn_b"][i].reshape(1, d),
            params["gin_W2"][i],
            params["bn_g"][i].reshape(1, d),
            params["bn_b"][i].reshape(1, d),
        )
        pools.append(pool)

    px = _pool_sum(x)
    zeros3 = jnp.zeros((3, d), jnp.float32)
    pall = jnp.concatenate([px[0:1]] + [p[0:1] for p in pools] + [zeros3], 0)
    pw = jnp.concatenate(list(params["pred_W"]), axis=0)
    pb = jnp.concatenate([b.reshape(1, d) for b in params["pred_b"]]
                         + [zeros3], axis=0)
    return _head(pall, pw, pb)


# speedup vs baseline: 9.1405x; 1.0461x over previous
import functools

import jax
import jax.numpy as jnp
from jax import lax
from jax.experimental import pallas as pl
from jax.experimental.pallas import tpu as pltpu
from jax.experimental.pallas import tpu_sc as plsc

BN_EPS = 1e-5
NC = 2
NS = 16
NW = NC * NS
CHUNK = 125
HIGH = lax.Precision.HIGHEST


def _sc_body(ngrp, nchg, h_hbm, src_hbm, dst_hbm, out_hbm,
             sidx, didx, rows, acc, gsem):
    n = out_hbm.shape[1]
    d = h_hbm.shape[1]
    cid = lax.axis_index("c")
    sid = lax.axis_index("s")
    wid = sid * NC + cid
    zr = 80
    ncopies = n // zr

    def zrow(r, carry):
        def zcol(c, carry2):
            rows[0, r, pl.ds(c * 16, 16)] = jnp.zeros((16,), jnp.float32)
            return carry2
        return lax.fori_loop(0, d // 16, zcol, carry)
    lax.fori_loop(0, zr, zrow, 0)

    for k in range(-(-ncopies // NS)):
        j = sid + k * NS

        @pl.when(j < ncopies)
        def _():
            pltpu.sync_copy(rows.at[0, pl.ds(0, zr)],
                            acc.at[pl.ds(pl.multiple_of(j * zr, 8), zr)])
    plsc.subcore_barrier()

    def gather(i, b):
        pltpu.async_copy(h_hbm.at[sidx.at[i]], rows.at[b], gsem.at[b])

    def drain(i, b):
        pltpu.make_async_copy(h_hbm.at[sidx.at[i]], rows.at[b],
                              gsem.at[b]).wait()
        pltpu.sync_copy(rows.at[b], acc.at[didx.at[i]], add=True)

    for g in range(ngrp):
        pltpu.sync_copy(src_hbm.at[wid, g], sidx)
        pltpu.sync_copy(dst_hbm.at[wid, g], didx)
        gather(0, 0)

        def step(j, carry):
            i0 = j * 2

            @pl.when(i0 + 1 < nchg)
            def _():
                gather(i0 + 1, 1)
            drain(i0, 0)

            @pl.when(i0 + 2 < nchg)
            def _():
                gather(i0 + 2, 0)

            @pl.when(i0 + 1 < nchg)
            def _():
                drain(i0 + 1, 1)
            return carry

        lax.fori_loop(0, (nchg + 1) // 2, step, 0)

    plsc.subcore_barrier()

    for k in range(-(-ncopies // NS)):
        j = sid + k * NS

        @pl.when(j < ncopies)
        def _():
            sl = pl.ds(pl.multiple_of(j * zr, 8), zr)
            pltpu.sync_copy(acc.at[sl], out_hbm.at[cid, sl])


def _sc_segment_sum(h, src4, dst4):
    n, d = h.shape
    _, ngrp, nchg, c = src4.shape
    mesh = plsc.VectorSubcoreMesh(core_axis_name="c", subcore_axis_name="s")
    f = pl.kernel(
        functools.partial(_sc_body, ngrp, nchg),
        out_type=jax.ShapeDtypeStruct((NC, n, d), jnp.float32),
        mesh=mesh,
        scratch_types=[
            pltpu.VMEM((nchg, c), jnp.int32),
            pltpu.VMEM((nchg, c), jnp.int32),
            pltpu.VMEM((2, c, d), jnp.float32),
            pltpu.VMEM_SHARED((n, d), jnp.float32),
            pltpu.SemaphoreType.DMA((2,)),
        ],
    )
    return f(h, src4, dst4)


def _mm_t(a, w):
    return lax.dot_general(a, w, (((1,), (1,)), ((), ())), precision=HIGH)


def _bn_scale_shift(s_ref, g_ref, b_ref, n):
    m = s_ref[pl.ds(0, 1), :] * (1.0 / n)
    ex2 = s_ref[pl.ds(1, 1), :] * (1.0 / n)
    v = ex2 - m * m
    scale = g_ref[...] * lax.rsqrt(v + BN_EPS)
    shift = b_ref[...] - m * scale
    return scale, shift


def _sumsq_blk(t):
    return jnp.concatenate(
        [jnp.sum(t, axis=0, keepdims=True),
         jnp.sum(t * t, axis=0, keepdims=True),
         jnp.zeros((6, t.shape[1]), jnp.float32)], axis=0)


def _fused_layer_body(n, mode, *refs):
    if mode == "first":
        (h_ref, p0_ref, p1_ref, w1_ref, g1_ref, b1_ref, w2_ref, g2_ref,
         b2_ref, hh_ref, pool_ref, px_ref, ts_ref, s1_ref, s2_ref) = refs
    elif mode == "last":
        (h_ref, p0_ref, p1_ref, w1_ref, g1_ref, b1_ref, w2_ref, g2_ref,
         b2_ref, pall_ref, pw_ref, pb_ref, res_ref,
         ts_ref, s1_ref, s2_ref, s3_ref) = refs
    else:
        (h_ref, p0_ref, p1_ref, w1_ref, g1_ref, b1_ref, w2_ref, g2_ref,
         b2_ref, hh_ref, pool_ref, ts_ref, s1_ref, s2_ref) = refs

    p = pl.program_id(0)
    i = pl.program_id(1)
    r = h_ref.shape[0]
    d = h_ref.shape[1]
    rows = pl.ds(i * r, r)

    @pl.when(p == 0)
    def _():
        hb = h_ref[...]
        t = _mm_t(hb + p0_ref[...] + p1_ref[...], w1_ref[...])
        ts_ref[rows, :] = t

        @pl.when(i == 0)
        def _():
            s1_ref[...] = jnp.zeros_like(s1_ref)
        s1_ref[...] += _sumsq_blk(t)
        if mode == "first":
            @pl.when(i == 0)
            def _():
                px_ref[...] = jnp.zeros_like(px_ref)
            px_ref[...] += jnp.concatenate(
                [jnp.sum(hb, axis=0, keepdims=True),
                 jnp.zeros((7, d), jnp.float32)], axis=0)

    @pl.when(p == 1)
    def _():
        scale, shift = _bn_scale_shift(s1_ref, g1_ref, b1_ref, n)
        u = jnp.maximum(ts_ref[rows, :] * scale + shift, 0.0)
        o = _mm_t(u, w2_ref[...])
        ts_ref[rows, :] = o

        @pl.when(i == 0)
        def _():
            s2_ref[...] = jnp.zeros_like(s2_ref)
        s2_ref[...] += _sumsq_blk(o)

    @pl.when(p == 2)
    def _():
        scale, shift = _bn_scale_shift(s2_ref, g2_ref, b2_ref, n)
        hh = jnp.maximum(ts_ref[rows, :] * scale + shift, 0.0)
        pblk = jnp.concatenate(
            [jnp.sum(hh, axis=0, keepdims=True),
             jnp.zeros((7, d), jnp.float32)], axis=0)
        if mode == "last":
            @pl.when(i == 0)
            def _():
                s3_ref[...] = jnp.zeros_like(s3_ref)
            s3_ref[...] += pblk
        else:
            hh_ref[...] = hh

            @pl.when(i == 0)
            def _():
                pool_ref[...] = jnp.zeros_like(pool_ref)
            pool_ref[...] += pblk

    if mode == "last":
        @pl.when((p == 3) & (i == 0))
        def _():
            acc = jnp.zeros((1, d), jnp.float32)
            for k in range(4):
                acc = (acc + _mm_t(pall_ref[pl.ds(k, 1), :],
                                   pw_ref[pl.ds(k * d, d), :])
                       + pb_ref[pl.ds(k, 1), :])
            acc = (acc + _mm_t(s3_ref[pl.ds(0, 1), :],
                               pw_ref[pl.ds(4 * d, d), :])
                   + pb_ref[pl.ds(4, 1), :])
            z = acc - jnp.max(acc, axis=-1, keepdims=True)
            res_ref[...] = z - jnp.log(
                jnp.sum(jnp.exp(z), axis=-1, keepdims=True))


def _phase_row_spec(r, d, ph):
    return pl.BlockSpec((r, d), lambda p, i: (jnp.where(p == ph, i, 0), 0))


def _pin_spec(shape):
    return pl.BlockSpec(shape, lambda p, i: tuple(0 for _ in shape))


def _tc_layer(h, p0, p1, w1, g1, b1, w2, g2, b2, mode,
              pall=None, pw=None, pb=None):
    n, d = h.shape
    r = 1000
    g = n // r
    f32 = jnp.float32
    in_specs = [_phase_row_spec(r, d, 0)] * 3 + [
        _pin_spec((d, d)), _pin_spec((1, d)), _pin_spec((1, d)),
        _pin_spec((d, d)), _pin_spec((1, d)), _pin_spec((1, d))]
    scratch = [pltpu.VMEM((n, d), f32), pltpu.VMEM((8, d), f32),
               pltpu.VMEM((8, d), f32)]
    args = [h, p0, p1, w1, g1, b1, w2, g2, b2]
    if mode == "last":
        in_specs += [_pin_spec((8, d)), _pin_spec((5 * d, d)),
                     _pin_spec((8, d))]
        args += [pall, pw, pb]
        out_specs = _pin_spec((1, d))
        out_shape = jax.ShapeDtypeStruct((1, d), f32)
        scratch.append(pltpu.VMEM((8, d), f32))
        nphase = 4
    else:
        out_specs = [_phase_row_spec(r, d, 2), _pin_spec((8, d))]
        out_shape = [jax.ShapeDtypeStruct((n, d), f32),
                     jax.ShapeDtypeStruct((8, d), f32)]
        if mode == "first":
            out_specs.append(_pin_spec((8, d)))
            out_shape.append(jax.ShapeDtypeStruct((8, d), f32))
        nphase = 3
    return pl.pallas_call(
        functools.partial(_fused_layer_body, n, mode),
        grid=(nphase, g),
        in_specs=in_specs,
        out_specs=out_specs,
        out_shape=out_shape,
        scratch_shapes=scratch,
    )(*args)


def kernel(x, edge_index, params):
    n, d = x.shape
    e = edge_index.shape[1]
    ngrp = 5
    nchg = e // (NW * CHUNK * ngrp)
    src4 = edge_index[0].reshape(NW, ngrp, nchg, CHUNK)
    dst4 = edge_index[1].reshape(NW, ngrp, nchg, CHUNK)

    pw = jnp.concatenate(list(params["pred_W"]), axis=0)
    pb = jnp.concatenate([b.reshape(1, d) for b in params["pred_b"]]
                         + [jnp.zeros((3, d), jnp.float32)], axis=0)

    h = x
    pools = []
    for i in range(4):
        lp = (params["gin_W1"][i],
              params["gin_bn_g"][i].reshape(1, d),
              params["gin_bn_b"][i].reshape(1, d),
              params["gin_W2"][i],
              params["bn_g"][i].reshape(1, d),
              params["bn_b"][i].reshape(1, d))
        agg = _sc_segment_sum(h, src4, dst4)
        if i == 0:
            h, pool, px = _tc_layer(h, agg[0], agg[1], *lp, "first")
            pools.append(px)
            pools.append(pool)
        elif i < 3:
            h, pool = _tc_layer(h, agg[0], agg[1], *lp, "mid")
            pools.append(pool)
        else:
            pall = jnp.concatenate(
                [p[0:1] for p in pools] + [jnp.zeros((4, d), jnp.float32)], 0)
            return _tc_layer(h, agg[0], agg[1], *lp, "last",
                             pall=pall, pw=pw, pb=pb)
